# baseline (speedup 1.0000x reference)
import functools

import jax
import jax.numpy as jnp
from jax import lax
from jax.experimental import pallas as pl
from jax.experimental.pallas import tpu as pltpu
from jax.experimental.pallas import tpu_sc as plsc

NC, NS = 2, 16
NW = NC * NS
BATCH, HIST, D = 4096, 50, 256
B = BATCH * HIST
CHUNK = 64
CPW = B // (NW * CHUNK)
NBUF = 4


@functools.partial(
    pl.kernel,
    out_type=jax.ShapeDtypeStruct((B, D), jnp.float32),
    mesh=plsc.VectorSubcoreMesh(core_axis_name="c", subcore_axis_name="s"),
    scratch_types=[
        pltpu.VMEM((CPW, CHUNK), jnp.int32),
        pltpu.VMEM((NBUF, CHUNK, D), jnp.float32),
    ]
    + [pltpu.SemaphoreType.DMA] * NBUF
    + [pltpu.SemaphoreType.DMA] * NBUF,
)
def _gather_kernel(table_hbm, idx_hbm, out_hbm, idx_v, rows_v, *sems):
    gsem, ssem = sems[:NBUF], sems[NBUF:]
    wid = lax.axis_index("s") * NC + lax.axis_index("c")
    base = wid * (CPW * CHUNK)
    pltpu.sync_copy(idx_hbm.at[wid], idx_v)

    def start_gather(c, b):
        pltpu.async_copy(table_hbm.at[idx_v.at[c]], rows_v.at[b], gsem[b])

    def wait_gather(b):
        pltpu.make_async_copy(
            table_hbm.at[idx_v.at[b]], rows_v.at[b], gsem[b]
        ).wait()

    def start_scatter(c, b):
        pltpu.async_copy(
            rows_v.at[b], out_hbm.at[pl.ds(base + c * CHUNK, CHUNK)], ssem[b]
        )

    def wait_scatter(b):
        pltpu.make_async_copy(
            rows_v.at[b], out_hbm.at[pl.ds(base, CHUNK)], ssem[b]
        ).wait()

    for b in range(NBUF):
        start_gather(b, b)

    def body(k, carry):
        for b in range(NBUF):
            c = k * NBUF + b
            wait_gather(b)
            start_scatter(c, b)
        for b in range(NBUF):
            wait_scatter(b)
            start_gather(k * NBUF + NBUF + b, b)
        return carry

    lax.fori_loop(0, CPW // NBUF - 1, body, 0)

    for b in range(NBUF):
        c = CPW - NBUF + b
        wait_gather(b)
        start_scatter(c, b)
    for b in range(NBUF):
        wait_scatter(b)


def kernel(actions, action_embeddings):
    idx = actions.reshape(NW, CPW, CHUNK).astype(jnp.int32)
    out = _gather_kernel(action_embeddings, idx)
    return out.reshape(BATCH, HIST, D)


# speedup vs baseline: 1.2351x; 1.2351x over previous
import functools

import jax
import jax.numpy as jnp
from jax import lax
from jax.experimental import pallas as pl
from jax.experimental.pallas import tpu as pltpu
from jax.experimental.pallas import tpu_sc as plsc

NC, NS = 2, 16
NW = NC * NS
BATCH, HIST, D = 4096, 50, 256
B = BATCH * HIST
ROWS = 74
CHUNK = 128
CPW = B // (NW * CHUNK)
NBUF = 2
NG = D // 16


@functools.partial(
    pl.kernel,
    out_type=jax.ShapeDtypeStruct((B, D), jnp.float32),
    mesh=plsc.VectorSubcoreMesh(core_axis_name="c", subcore_axis_name="s"),
    scratch_types=[
        pltpu.VMEM((ROWS, D), jnp.float32),
        pltpu.VMEM((CPW, CHUNK), jnp.int32),
        pltpu.VMEM((NBUF, CHUNK, D), jnp.float32),
    ]
    + [pltpu.SemaphoreType.DMA] * NBUF,
)
def _gather_kernel(table_hbm, idx_hbm, out_hbm, table_v, idx_v, rows_v, *ssem):
    wid = lax.axis_index("s") * NC + lax.axis_index("c")
    base = wid * (CPW * CHUNK)
    pltpu.sync_copy(table_hbm, table_v)
    pltpu.sync_copy(idx_hbm.at[wid], idx_v)

    def build(c, b):
        def group_body(g, carry):
            ivec = idx_v[c, pl.ds(16 * g, 16)]
            for l in range(16):
                a = ivec[l]
                i = 16 * g + l
                for j in range(NG):
                    rows_v[b, i, pl.ds(16 * j, 16)] = table_v[
                        a, pl.ds(16 * j, 16)
                    ]
            return carry

        lax.fori_loop(0, CHUNK // 16, group_body, 0)

    def start_scatter(c, b):
        pltpu.async_copy(
            rows_v.at[b], out_hbm.at[pl.ds(base + c * CHUNK, CHUNK)], ssem[b]
        )

    def wait_scatter(b):
        pltpu.make_async_copy(
            rows_v.at[b], out_hbm.at[pl.ds(base, CHUNK)], ssem[b]
        ).wait()

    for b in range(NBUF):
        build(b, b)
        start_scatter(b, b)

    def body(k, carry):
        for b in range(NBUF):
            c = k * NBUF + b
            wait_scatter(b)
            build(c, b)
            start_scatter(c, b)
        return carry

    lax.fori_loop(1, CPW // NBUF, body, 0)

    for b in range(NBUF):
        wait_scatter(b)


def kernel(actions, action_embeddings):
    idx = actions.reshape(NW, CPW, CHUNK).astype(jnp.int32)
    out = _gather_kernel(action_embeddings, idx)
    return out.reshape(BATCH, HIST, D)


# speedup vs baseline: 1.8529x; 1.5002x over previous
import functools

import jax
import jax.numpy as jnp
from jax import lax
from jax.experimental import pallas as pl
from jax.experimental.pallas import tpu as pltpu
from jax.experimental.pallas import tpu_sc as plsc

NC, NS = 2, 16
NW = NC * NS
BATCH, HIST, D = 4096, 50, 256
ROWS = 74
CPW = BATCH // NW
NBUF = 2
NG = D // 16
GROUPS = ((0, range(16)), (16, range(16)), (32, range(16)), (34, (14, 15)))


@functools.partial(
    pl.kernel,
    out_type=jax.ShapeDtypeStruct((BATCH, HIST, D), jnp.float32),
    mesh=plsc.VectorSubcoreMesh(core_axis_name="c", subcore_axis_name="s"),
    scratch_types=[
        pltpu.VMEM((ROWS, D), jnp.float32),
        pltpu.VMEM((CPW, HIST), jnp.int32),
        pltpu.VMEM((NBUF, HIST, D), jnp.float32),
    ]
    + [pltpu.SemaphoreType.DMA] * NBUF,
)
def _gather_kernel(table_hbm, idx_hbm, out_hbm, table_v, idx_v, rows_v, *ssem):
    wid = lax.axis_index("s") * NC + lax.axis_index("c")
    base = wid * CPW
    pltpu.sync_copy(table_hbm, table_v)
    pltpu.sync_copy(idx_hbm.at[wid], idx_v)

    def build(c, b):
        for off, lanes in GROUPS:
            ivec = idx_v[c, pl.ds(off, 16)]
            for l in lanes:
                i = off + l
                a = ivec[l]
                vals = [table_v[a, pl.ds(16 * j, 16)] for j in range(NG)]
                for j in range(NG):
                    rows_v[b, i, pl.ds(16 * j, 16)] = vals[j]

    def start_scatter(c, b):
        pltpu.async_copy(rows_v.at[b], out_hbm.at[base + c], ssem[b])

    def wait_scatter(b):
        pltpu.make_async_copy(rows_v.at[b], out_hbm.at[base], ssem[b]).wait()

    for b in range(NBUF):
        build(b, b)
        start_scatter(b, b)

    def body(k, carry):
        for b in range(NBUF):
            c = k * NBUF + b
            wait_scatter(b)
            build(c, b)
            start_scatter(c, b)
        return carry

    lax.fori_loop(1, CPW // NBUF, body, 0)

    for b in range(NBUF):
        wait_scatter(b)


def kernel(actions, action_embeddings):
    idx = actions.reshape(NW, CPW, HIST).astype(jnp.int32)
    return _gather_kernel(action_embeddings, idx)
